# baseline (speedup 1.0000x reference)
import functools

import jax
import jax.numpy as jnp
from jax import lax
from jax.experimental import pallas as pl
from jax.experimental.pallas import tpu as pltpu
from jax.experimental.pallas import tpu_sc as plsc

N = 10000
NP = 10240
E = 320000
NC, NS = 2, 16
CHUNK = 128
EP = 2560 * CHUNK
H = 128
NH = NP // NC
RPH = NH // NS
RPS = NP // NS
CPS = EP // CHUNK // NS
ZROW = N

_mesh = plsc.VectorSubcoreMesh(core_axis_name="c", subcore_axis_name="s")
_cp = pltpu.CompilerParams(use_tc_tiling_on_sc=False)


@functools.partial(
    pl.kernel,
    out_type=jax.ShapeDtypeStruct((NC, NP, 16), jnp.float32),
    mesh=_mesh,
    scratch_types=[
        pltpu.VMEM((CPS // 2, CHUNK), jnp.int32),
        pltpu.VMEM((CHUNK, 16), jnp.float32),
        pltpu.VMEM_SHARED((NP, 16), jnp.float32),
        pltpu.SemaphoreType.DMA,
    ],
    compiler_params=_cp,
)
def _sc_deg(dst_hbm, ones_hbm, zeros_hbm, out_hbm, dst_v, ones_v, acc, sem):
    cid = lax.axis_index("c")
    sid = lax.axis_index("s")
    cpw = CPS // 2
    wchunk = (cid * NS + sid) * cpw
    pltpu.async_copy(zeros_hbm, acc.at[pl.ds(sid * RPS, RPS)], sem).wait()
    pltpu.async_copy(ones_hbm, ones_v, sem).wait()
    pltpu.async_copy(dst_hbm.at[pl.ds(wchunk, cpw)], dst_v, sem).wait()
    plsc.subcore_barrier()

    @pl.loop(0, cpw)
    def _(j):
        pltpu.sync_copy(ones_v, acc.at[dst_v.at[j]], add=True)

    plsc.subcore_barrier()
    pltpu.sync_copy(acc.at[pl.ds(sid * RPS, RPS)],
                    out_hbm.at[cid, pl.ds(sid * RPS, RPS)])


@functools.partial(
    pl.kernel,
    out_type=jax.ShapeDtypeStruct((NC, NH, H), jnp.float32),
    mesh=_mesh,
    scratch_types=[
        pltpu.VMEM((CPS, CHUNK), jnp.int32),
        pltpu.VMEM((CPS, CHUNK), jnp.int32),
        *([pltpu.VMEM((CHUNK, H), jnp.float32)] * 2),
        pltpu.VMEM_SHARED((NH, H), jnp.float32),
        *([pltpu.SemaphoreType.DMA] * 2),
        *([pltpu.SemaphoreType.DMA] * 2),
    ],
    compiler_params=_cp,
)
def _sc_agg(g_hbm, src_hbm, dst_hbm, zeros_hbm, out_hbm,
            src_v, dst_v, r0, r1, acc,
            g0, g1, s0, s1):
    rows = (r0, r1)
    gsem = (g0, g1)
    ssem = (s0, s1)
    cid = lax.axis_index("c")
    sid = lax.axis_index("s")
    wchunk = sid * CPS
    pltpu.async_copy(zeros_hbm, acc.at[pl.ds(sid * RPH, RPH)], g0).wait()
    pltpu.async_copy(src_hbm.at[cid, pl.ds(wchunk, CPS)], src_v, g0).wait()
    pltpu.async_copy(dst_hbm.at[cid, pl.ds(wchunk, CPS)], dst_v, g1).wait()
    plsc.subcore_barrier()

    for b in range(2):
        pltpu.async_copy(g_hbm.at[src_v.at[b]], rows[b], gsem[b])

    @pl.loop(0, CPS - 2, step=2)
    def _(j):
        for b in range(2):
            pltpu.make_async_copy(g_hbm.at[src_v.at[j + b]], rows[b],
                                  gsem[b]).wait()
            pltpu.async_copy(rows[b], acc.at[dst_v.at[j + b]], ssem[b],
                             add=True)
        for b in range(2):
            pltpu.make_async_copy(rows[b], acc.at[dst_v.at[j + b]],
                                  ssem[b]).wait()
            pltpu.async_copy(g_hbm.at[src_v.at[j + 2 + b]], rows[b], gsem[b])

    for b in range(2):
        c = CPS - 2 + b
        pltpu.make_async_copy(g_hbm.at[src_v.at[c]], rows[b], gsem[b]).wait()
        pltpu.async_copy(rows[b], acc.at[dst_v.at[c]], ssem[b], add=True)
    for b in range(2):
        pltpu.make_async_copy(rows[b], acc.at[dst_v.at[CPS - 2 + b]],
                              ssem[b]).wait()

    plsc.subcore_barrier()
    pltpu.sync_copy(acc.at[pl.ds(sid * RPH, RPH)],
                    out_hbm.at[cid, pl.ds(sid * RPH, RPH)])


def _dinv_of(degp_ref):
    deg = 1.0 + degp_ref[0, :, 0:1] + degp_ref[1, :, 0:1]
    return lax.rsqrt(deg)


def _leaky(u):
    return jnp.where(u >= 0, u, 0.01 * u)


def _rowmask():
    return (lax.broadcasted_iota(jnp.int32, (NP, 1), 0) < N).astype(jnp.float32)


def _tc_g0_body(x_ref, w_ref, degp_ref, out_ref):
    dinv = _dinv_of(degp_ref)
    out_ref[...] = _rowmask() * (
        dinv * jnp.dot(x_ref[...], w_ref[...],
                       preferred_element_type=jnp.float32))


def _tc_mid_body(agg_ref, gprev_ref, degp_ref, b_ref, w_ref, out_ref):
    dinv = _dinv_of(degp_ref)
    u = dinv * (agg_ref[...] + gprev_ref[...]) + b_ref[...]
    v = _leaky(u)
    out_ref[...] = _rowmask() * (
        dinv * jnp.dot(v, w_ref[...], preferred_element_type=jnp.float32))


def _tc_final_body(agg_ref, g_ref, degp_ref, b_ref, batch_ref, xs_ref,
                   wp_ref, ws_ref, bl1_ref, wl2_ref, bl2_ref, out_ref):
    dinv = _dinv_of(degp_ref)
    hidden = _leaky(dinv * (agg_ref[...] + g_ref[...]) + b_ref[...])
    seg = lax.broadcasted_iota(jnp.int32, (1, 16), 1)
    mask = (batch_ref[...] == seg).astype(jnp.float32)
    pooled = lax.dot_general(mask, hidden, (((0,), (0,)), ((), ())),
                             preferred_element_type=jnp.float32)
    cnt = lax.dot_general(mask, jnp.ones((NP, 1), jnp.float32),
                          (((0,), (0,)), ((), ())),
                          preferred_element_type=jnp.float32)
    pooled = pooled / jnp.maximum(cnt, 1.0)
    h1 = (jnp.dot(pooled, wp_ref[...], preferred_element_type=jnp.float32)
          + jnp.dot(xs_ref[...], ws_ref[...], preferred_element_type=jnp.float32)
          + bl1_ref[...])
    h1 = _leaky(h1)
    out_ref[...] = jnp.dot(h1, wl2_ref[...],
                           preferred_element_type=jnp.float32) + bl2_ref[...]


def _tc(body, out_shape, *args):
    return pl.pallas_call(
        body, out_shape=jax.ShapeDtypeStruct(out_shape, jnp.float32))(*args)


def kernel(x, edge_index, x_scalar, batch_index,
           W0, b0, W1, b1, W2, b2, W3, b3, Wl1, bl1, Wl2, bl2):
    src = edge_index[0].astype(jnp.int32)
    dst = edge_index[1].astype(jnp.int32)
    pad = EP - E
    dead = jnp.full((pad,), NP - 1, jnp.int32)
    srcp = jnp.concatenate([src, dead])
    dstp = jnp.concatenate([dst, dead])
    dst2d = dstp.reshape(EP // CHUNK, CHUNK)

    spread = jnp.arange(EP, dtype=jnp.int32) % NH
    src_cs, dst_cs = [], []
    for c in range(NC):
        mine = (dstp >= c * NH) & (dstp < (c + 1) * NH)
        src_cs.append(jnp.where(mine, srcp, ZROW))
        dst_cs.append(jnp.where(mine, dstp - c * NH, spread))
    src_pc = jnp.stack(src_cs).reshape(NC, EP // CHUNK, CHUNK)
    dst_pc = jnp.stack(dst_cs).reshape(NC, EP // CHUNK, CHUNK)

    x_pad = jnp.pad(x, ((0, NP - N), (0, 0)))
    batch_pad = jnp.pad(batch_index.astype(jnp.int32), (0, NP - N),
                        constant_values=16).reshape(NP, 1)
    zerosH = jnp.zeros((RPH, H), jnp.float32)
    zeros16 = jnp.zeros((RPS, 16), jnp.float32)
    ones16 = jnp.ones((CHUNK, 16), jnp.float32)

    wp = Wl1[:H]
    ws = jnp.pad(Wl1[H:], ((0, H - 4), (0, 0)))
    xs_pad = jnp.pad(x_scalar, ((0, 0), (0, H - 4)))
    wl2 = jnp.pad(Wl2, ((0, 0), (0, H - 1)))
    bl2p = jnp.pad(bl2.reshape(1, 1).astype(jnp.float32),
                   ((0, 0), (0, H - 1)))

    degp = _sc_deg(dst2d, ones16, zeros16)

    g = _tc(_tc_g0_body, (NP, H), x_pad, W0, degp)
    for b_prev, W_next in ((b0, W1), (b1, W2), (b2, W3)):
        agg = _sc_agg(g, src_pc, dst_pc, zerosH).reshape(NP, H)
        g = _tc(_tc_mid_body, (NP, H), agg, g, degp,
                b_prev.reshape(1, H), W_next)
    agg = _sc_agg(g, src_pc, dst_pc, zerosH).reshape(NP, H)

    out = _tc(_tc_final_body, (16, H), agg, g, degp, b3.reshape(1, H),
              batch_pad, xs_pad, wp, ws, bl1.reshape(1, H), wl2, bl2p)
    return out[:, :1]


# speedup vs baseline: 20.3456x; 20.3456x over previous
import functools

import jax
import jax.numpy as jnp
from jax import lax
from jax.experimental import pallas as pl
from jax.experimental.pallas import tpu as pltpu
from jax.experimental.pallas import tpu_sc as plsc

N = 10000
NP = 10240
E = 320000
NC, NS = 2, 16
CHUNK = 128
EP = 2560 * CHUNK
H = 128
NH = NP // NC
RPH = NH // NS
RPS = NP // NS
CPS = EP // CHUNK // NS
ZROW = N

_mesh = plsc.VectorSubcoreMesh(core_axis_name="c", subcore_axis_name="s")
_cp = pltpu.CompilerParams(use_tc_tiling_on_sc=False)


@functools.partial(
    pl.kernel,
    out_type=jax.ShapeDtypeStruct((NC, NP, 16), jnp.float32),
    mesh=_mesh,
    scratch_types=[
        pltpu.VMEM((CPS // 2, CHUNK), jnp.int32),
        pltpu.VMEM((CHUNK, 16), jnp.float32),
        pltpu.VMEM_SHARED((NP, 16), jnp.float32),
        pltpu.SemaphoreType.DMA,
    ],
    compiler_params=_cp,
)
def _sc_deg(dst_hbm, ones_hbm, zeros_hbm, out_hbm, dst_v, ones_v, acc, sem):
    cid = lax.axis_index("c")
    sid = lax.axis_index("s")
    cpw = CPS // 2
    wchunk = (cid * NS + sid) * cpw
    pltpu.async_copy(zeros_hbm, acc.at[pl.ds(sid * RPS, RPS)], sem).wait()
    pltpu.async_copy(ones_hbm, ones_v, sem).wait()
    pltpu.async_copy(dst_hbm.at[pl.ds(wchunk, cpw)], dst_v, sem).wait()
    plsc.subcore_barrier()

    @pl.loop(0, cpw)
    def _(j):
        pltpu.sync_copy(ones_v, acc.at[dst_v.at[j]], add=True)

    plsc.subcore_barrier()
    pltpu.sync_copy(acc.at[pl.ds(sid * RPS, RPS)],
                    out_hbm.at[cid, pl.ds(sid * RPS, RPS)])


@functools.partial(
    pl.kernel,
    out_type=jax.ShapeDtypeStruct((NC, NH, H), jnp.float32),
    mesh=_mesh,
    scratch_types=[
        pltpu.VMEM((CPS, CHUNK), jnp.int32),
        pltpu.VMEM((CPS, CHUNK), jnp.int32),
        *([pltpu.VMEM((CHUNK, H), jnp.float32)] * 2),
        pltpu.VMEM_SHARED((NH, H), jnp.float32),
        *([pltpu.SemaphoreType.DMA] * 2),
        *([pltpu.SemaphoreType.DMA] * 2),
    ],
    compiler_params=_cp,
)
def _sc_agg(g_hbm, src_hbm, dst_hbm, zeros_hbm, out_hbm,
            src_v, dst_v, r0, r1, acc,
            g0, g1, s0, s1):
    rows = (r0, r1)
    gsem = (g0, g1)
    ssem = (s0, s1)
    cid = lax.axis_index("c")
    sid = lax.axis_index("s")
    wchunk = sid * CPS
    pltpu.async_copy(zeros_hbm, acc.at[pl.ds(sid * RPH, RPH)], g0).wait()
    pltpu.async_copy(src_hbm.at[cid, pl.ds(wchunk, CPS)], src_v, g0).wait()
    pltpu.async_copy(dst_hbm.at[cid, pl.ds(wchunk, CPS)], dst_v, g1).wait()
    plsc.subcore_barrier()

    for b in range(2):
        pltpu.async_copy(g_hbm.at[src_v.at[b]], rows[b], gsem[b])

    @pl.loop(0, CPS - 2, step=2)
    def _(j):
        for b in range(2):
            pltpu.make_async_copy(g_hbm.at[src_v.at[j + b]], rows[b],
                                  gsem[b]).wait()
            pltpu.async_copy(rows[b], acc.at[dst_v.at[j + b]], ssem[b],
                             add=True)
        for b in range(2):
            pltpu.make_async_copy(rows[b], acc.at[dst_v.at[j + b]],
                                  ssem[b]).wait()
            pltpu.async_copy(g_hbm.at[src_v.at[j + 2 + b]], rows[b], gsem[b])

    for b in range(2):
        c = CPS - 2 + b
        pltpu.make_async_copy(g_hbm.at[src_v.at[c]], rows[b], gsem[b]).wait()
        pltpu.async_copy(rows[b], acc.at[dst_v.at[c]], ssem[b], add=True)
    for b in range(2):
        pltpu.make_async_copy(rows[b], acc.at[dst_v.at[CPS - 2 + b]],
                              ssem[b]).wait()

    plsc.subcore_barrier()
    pltpu.sync_copy(acc.at[pl.ds(sid * RPH, RPH)],
                    out_hbm.at[cid, pl.ds(sid * RPH, RPH)])


def _dinv_of(degp_ref):
    deg = 1.0 + degp_ref[0, :, 0:1] + degp_ref[1, :, 0:1]
    return lax.rsqrt(deg)


def _leaky(u):
    return jnp.where(u >= 0, u, 0.01 * u)


def _rowmask():
    return (lax.broadcasted_iota(jnp.int32, (NP, 1), 0) < N).astype(jnp.float32)


def _tc_g0_body(x_ref, w_ref, degp_ref, out_ref):
    dinv = _dinv_of(degp_ref)
    out_ref[...] = _rowmask() * (
        dinv * jnp.dot(x_ref[...], w_ref[...],
                       preferred_element_type=jnp.float32))


def _tc_mid_body(agg_ref, gprev_ref, degp_ref, b_ref, w_ref, out_ref):
    dinv = _dinv_of(degp_ref)
    u = dinv * (agg_ref[...] + gprev_ref[...]) + b_ref[...]
    v = _leaky(u)
    out_ref[...] = _rowmask() * (
        dinv * jnp.dot(v, w_ref[...], preferred_element_type=jnp.float32))


def _tc_final_body(agg_ref, g_ref, degp_ref, b_ref, batch_ref, xs_ref,
                   wp_ref, ws_ref, bl1_ref, wl2_ref, bl2_ref, out_ref):
    dinv = _dinv_of(degp_ref)
    hidden = _leaky(dinv * (agg_ref[...] + g_ref[...]) + b_ref[...])
    seg = lax.broadcasted_iota(jnp.int32, (1, 16), 1)
    mask = (batch_ref[...] == seg).astype(jnp.float32)
    pooled = lax.dot_general(mask, hidden, (((0,), (0,)), ((), ())),
                             preferred_element_type=jnp.float32)
    cnt = lax.dot_general(mask, jnp.ones((NP, 1), jnp.float32),
                          (((0,), (0,)), ((), ())),
                          preferred_element_type=jnp.float32)
    pooled = pooled / jnp.maximum(cnt, 1.0)
    h1 = (jnp.dot(pooled, wp_ref[...], preferred_element_type=jnp.float32)
          + jnp.dot(xs_ref[...], ws_ref[...], preferred_element_type=jnp.float32)
          + bl1_ref[...])
    h1 = _leaky(h1)
    out_ref[...] = jnp.dot(h1, wl2_ref[...],
                           preferred_element_type=jnp.float32) + bl2_ref[...]


def _tc(body, out_shape, *args):
    return pl.pallas_call(
        body, out_shape=jax.ShapeDtypeStruct(out_shape, jnp.float32))(*args)


def kernel(x, edge_index, x_scalar, batch_index,
           W0, b0, W1, b1, W2, b2, W3, b3, Wl1, bl1, Wl2, bl2):
    src = edge_index[0].astype(jnp.int32)
    dst = edge_index[1].astype(jnp.int32)
    pad = EP - E
    dead = jnp.full((pad,), NP - 1, jnp.int32)
    srcp = jnp.concatenate([src, dead])
    dstp = jnp.concatenate([dst, dead])
    dst2d = dstp.reshape(EP // CHUNK, CHUNK)

    eid = jnp.arange(EP, dtype=jnp.int32)
    spread = eid % NH
    zspread = ZROW + eid % (NP - N)
    src_cs, dst_cs = [], []
    for c in range(NC):
        mine = (dstp >= c * NH) & (dstp < (c + 1) * NH)
        src_cs.append(jnp.where(mine, srcp, zspread))
        dst_cs.append(jnp.where(mine, dstp - c * NH, spread))
    src_pc = jnp.stack(src_cs).reshape(NC, EP // CHUNK, CHUNK)
    dst_pc = jnp.stack(dst_cs).reshape(NC, EP // CHUNK, CHUNK)

    x_pad = jnp.pad(x, ((0, NP - N), (0, 0)))
    batch_pad = jnp.pad(batch_index.astype(jnp.int32), (0, NP - N),
                        constant_values=16).reshape(NP, 1)
    zerosH = jnp.zeros((RPH, H), jnp.float32)
    zeros16 = jnp.zeros((RPS, 16), jnp.float32)
    ones16 = jnp.ones((CHUNK, 16), jnp.float32)

    wp = Wl1[:H]
    ws = jnp.pad(Wl1[H:], ((0, H - 4), (0, 0)))
    xs_pad = jnp.pad(x_scalar, ((0, 0), (0, H - 4)))
    wl2 = jnp.pad(Wl2, ((0, 0), (0, H - 1)))
    bl2p = jnp.pad(bl2.reshape(1, 1).astype(jnp.float32),
                   ((0, 0), (0, H - 1)))

    degp = _sc_deg(dst2d, ones16, zeros16)

    g = _tc(_tc_g0_body, (NP, H), x_pad, W0, degp)
    for b_prev, W_next in ((b0, W1), (b1, W2), (b2, W3)):
        agg = _sc_agg(g, src_pc, dst_pc, zerosH).reshape(NP, H)
        g = _tc(_tc_mid_body, (NP, H), agg, g, degp,
                b_prev.reshape(1, H), W_next)
    agg = _sc_agg(g, src_pc, dst_pc, zerosH).reshape(NP, H)

    out = _tc(_tc_final_body, (16, H), agg, g, degp, b3.reshape(1, H),
              batch_pad, xs_pad, wp, ws, bl1.reshape(1, H), wl2, bl2p)
    return out[:, :1]


# speedup vs baseline: 31.9730x; 1.5715x over previous
import functools

import jax
import jax.numpy as jnp
from jax import lax
from jax.experimental import pallas as pl
from jax.experimental.pallas import tpu as pltpu
from jax.experimental.pallas import tpu_sc as plsc

N = 10000
NP = 10240
E = 320000
NC, NS = 2, 16
CHUNK = 128
CPW = 80
EP = NC * NS * CPW * CHUNK
RPS = NP // NS
H = 128

_mesh = plsc.VectorSubcoreMesh(core_axis_name="c", subcore_axis_name="s")
_cp = pltpu.CompilerParams(use_tc_tiling_on_sc=False)


@functools.partial(
    pl.kernel,
    out_type=jax.ShapeDtypeStruct((NC, NP, 16), jnp.float32),
    mesh=_mesh,
    scratch_types=[
        pltpu.VMEM((CPW, CHUNK), jnp.int32),
        pltpu.VMEM((CHUNK, 16), jnp.float32),
        pltpu.VMEM_SHARED((NP, 16), jnp.float32),
        pltpu.SemaphoreType.DMA,
    ],
    compiler_params=_cp,
)
def _sc_deg(dst_hbm, ones_hbm, zeros_hbm, out_hbm, dst_v, ones_v, acc, sem):
    cid = lax.axis_index("c")
    sid = lax.axis_index("s")
    wchunk = (cid * NS + sid) * CPW
    pltpu.async_copy(zeros_hbm.at[pl.ds(sid * RPS, RPS)],
                     acc.at[pl.ds(sid * RPS, RPS)], sem).wait()
    pltpu.async_copy(ones_hbm, ones_v, sem).wait()
    pltpu.async_copy(dst_hbm.at[pl.ds(wchunk, CPW)], dst_v, sem).wait()
    plsc.subcore_barrier()

    @pl.loop(0, CPW)
    def _(j):
        pltpu.sync_copy(ones_v, acc.at[dst_v.at[j]], add=True)

    plsc.subcore_barrier()
    pltpu.sync_copy(acc.at[pl.ds(sid * RPS, RPS)],
                    out_hbm.at[cid, pl.ds(sid * RPS, RPS)])


HF = H // NC
CPS = EP // CHUNK // NS


@functools.partial(
    pl.kernel,
    out_type=jax.ShapeDtypeStruct((NC, NP, HF), jnp.float32),
    mesh=_mesh,
    scratch_types=[
        pltpu.VMEM((CPS, CHUNK), jnp.int32),
        pltpu.VMEM((CPS, CHUNK), jnp.int32),
        *([pltpu.VMEM((CHUNK, HF), jnp.float32)] * 4),
        pltpu.VMEM_SHARED((NP, HF), jnp.float32),
        *([pltpu.SemaphoreType.DMA] * 4),
        *([pltpu.SemaphoreType.DMA] * 4),
    ],
    compiler_params=_cp,
)
def _sc_agg(g_hbm, src_hbm, dst_hbm, zeros_hbm, out_hbm,
            src_v, dst_v, r0, r1, r2, r3, acc,
            g0, g1, g2, g3, s0, s1, s2, s3):
    rows = (r0, r1, r2, r3)
    gsem = (g0, g1, g2, g3)
    ssem = (s0, s1, s2, s3)
    cid = lax.axis_index("c")
    sid = lax.axis_index("s")
    wchunk = sid * CPS
    gh = g_hbm.at[cid]
    pltpu.async_copy(zeros_hbm.at[pl.ds(sid * RPS, RPS)],
                     acc.at[pl.ds(sid * RPS, RPS)], g0).wait()
    pltpu.async_copy(src_hbm.at[pl.ds(wchunk, CPS)], src_v, g0).wait()
    pltpu.async_copy(dst_hbm.at[pl.ds(wchunk, CPS)], dst_v, g1).wait()
    plsc.subcore_barrier()

    for b in range(4):
        pltpu.async_copy(gh.at[src_v.at[b]], rows[b], gsem[b])

    @pl.loop(0, CPS - 4, step=4)
    def _(j):
        for b in range(4):
            pltpu.make_async_copy(gh.at[src_v.at[j + b]], rows[b],
                                  gsem[b]).wait()
            pltpu.async_copy(rows[b], acc.at[dst_v.at[j + b]], ssem[b],
                             add=True)
        for b in range(4):
            pltpu.make_async_copy(rows[b], acc.at[dst_v.at[j + b]],
                                  ssem[b]).wait()
            pltpu.async_copy(gh.at[src_v.at[j + 4 + b]], rows[b], gsem[b])

    for b in range(4):
        c = CPS - 4 + b
        pltpu.make_async_copy(gh.at[src_v.at[c]], rows[b], gsem[b]).wait()
        pltpu.async_copy(rows[b], acc.at[dst_v.at[c]], ssem[b], add=True)
    for b in range(4):
        pltpu.make_async_copy(rows[b], acc.at[dst_v.at[CPS - 4 + b]],
                              ssem[b]).wait()

    plsc.subcore_barrier()
    pltpu.sync_copy(acc.at[pl.ds(sid * RPS, RPS)],
                    out_hbm.at[cid, pl.ds(sid * RPS, RPS)])


def _dinv_of(degp_ref):
    deg = 1.0 + degp_ref[0, :, 0:1] + degp_ref[1, :, 0:1]
    return lax.rsqrt(deg)


def _leaky(u):
    return jnp.where(u >= 0, u, 0.01 * u)


def _split(res, out_ref):
    out_ref[0] = res[:, :HF]
    out_ref[1] = res[:, HF:]


def _unsplit(ref):
    return jnp.concatenate([ref[0], ref[1]], axis=1)


def _tc_g0_body(x_ref, w_ref, degp_ref, out_ref):
    dinv = _dinv_of(degp_ref)
    _split(dinv * jnp.dot(x_ref[...], w_ref[...],
                          preferred_element_type=jnp.float32), out_ref)


def _tc_mid_body(aggp_ref, gprev_ref, degp_ref, b_ref, w_ref, out_ref):
    dinv = _dinv_of(degp_ref)
    u = dinv * (_unsplit(aggp_ref) + _unsplit(gprev_ref)) + b_ref[...]
    v = _leaky(u)
    _split(dinv * jnp.dot(v, w_ref[...],
                          preferred_element_type=jnp.float32), out_ref)


def _tc_final_body(aggp_ref, g_ref, degp_ref, b_ref, batch_ref, xs_ref,
                   wp_ref, ws_ref, bl1_ref, wl2_ref, bl2_ref, out_ref):
    dinv = _dinv_of(degp_ref)
    hidden = _leaky(dinv * (_unsplit(aggp_ref) + _unsplit(g_ref)) + b_ref[...])
    seg = lax.broadcasted_iota(jnp.int32, (1, 16), 1)
    mask = (batch_ref[...] == seg).astype(jnp.float32)
    pooled = lax.dot_general(mask, hidden, (((0,), (0,)), ((), ())),
                             preferred_element_type=jnp.float32)
    cnt = lax.dot_general(mask, jnp.ones((NP, 1), jnp.float32),
                          (((0,), (0,)), ((), ())),
                          preferred_element_type=jnp.float32)
    pooled = pooled / jnp.maximum(cnt, 1.0)
    h1 = (jnp.dot(pooled, wp_ref[...], preferred_element_type=jnp.float32)
          + jnp.dot(xs_ref[...], ws_ref[...], preferred_element_type=jnp.float32)
          + bl1_ref[...])
    h1 = _leaky(h1)
    out_ref[...] = jnp.dot(h1, wl2_ref[...],
                           preferred_element_type=jnp.float32) + bl2_ref[...]


def _tc(body, out_shape, *args):
    return pl.pallas_call(
        body, out_shape=jax.ShapeDtypeStruct(out_shape, jnp.float32))(*args)


def kernel(x, edge_index, x_scalar, batch_index,
           W0, b0, W1, b1, W2, b2, W3, b3, Wl1, bl1, Wl2, bl2):
    src = edge_index[0].astype(jnp.int32)
    dst = edge_index[1].astype(jnp.int32)
    pad = EP - E
    dead = jnp.full((pad,), NP - 1, jnp.int32)
    src2d = jnp.concatenate([src, dead]).reshape(EP // CHUNK, CHUNK)
    dst2d = jnp.concatenate([dst, dead]).reshape(EP // CHUNK, CHUNK)

    x_pad = jnp.pad(x, ((0, NP - N), (0, 0)))
    batch_pad = jnp.pad(batch_index.astype(jnp.int32), (0, NP - N),
                        constant_values=16).reshape(NP, 1)
    zerosH = jnp.zeros((NP, HF), jnp.float32)
    zeros16 = jnp.zeros((NP, 16), jnp.float32)
    ones16 = jnp.ones((CHUNK, 16), jnp.float32)

    wp = Wl1[:H]
    ws = jnp.pad(Wl1[H:], ((0, H - 4), (0, 0)))
    xs_pad = jnp.pad(x_scalar, ((0, 0), (0, H - 4)))
    wl2 = jnp.pad(Wl2, ((0, 0), (0, H - 1)))
    bl2p = jnp.pad(bl2.reshape(1, 1).astype(jnp.float32),
                   ((0, 0), (0, H - 1)))

    degp = _sc_deg(dst2d, ones16, zeros16)

    g = _tc(_tc_g0_body, (NC, NP, HF), x_pad, W0, degp)
    for b_prev, W_next in ((b0, W1), (b1, W2), (b2, W3)):
        aggp = _sc_agg(g, src2d, dst2d, zerosH)
        g = _tc(_tc_mid_body, (NC, NP, HF), aggp, g, degp,
                b_prev.reshape(1, H), W_next)
    aggp = _sc_agg(g, src2d, dst2d, zerosH)

    out = _tc(_tc_final_body, (16, H), aggp, g, degp, b3.reshape(1, H),
              batch_pad, xs_pad, wp, ws, bl1.reshape(1, H), wl2, bl2p)
    return out[:, :1]
